# baseline (speedup 1.0000x reference)
import jax
import jax.numpy as jnp
from jax.experimental import pallas as pl
from jax.experimental.pallas import tpu as pltpu

ORIGINAL_DIM = 4096
PROJ_DIM = 1024
C = 4
BATCH = 2048

BM = 512
BK = 512


def _sjlt_kernel(idx_ref, sign_ref, x_ref, o_ref, s_ref):
    i = pl.program_id(0)
    k = pl.program_id(1)

    @pl.when(i == 0)
    def _build_s_chunk():
        idx = idx_ref[...]
        sign = sign_ref[...]
        p = jax.lax.broadcasted_iota(jnp.int32, (BK, PROJ_DIM), 1)
        acc = jnp.zeros((BK, PROJ_DIM), jnp.float32)
        for j in range(C):
            acc += jnp.where(idx[:, j][:, None] == p, sign[:, j][:, None],
                             0.0)
        s_ref[pl.ds(k * BK, BK), :] = acc.astype(jnp.bfloat16)

    partial = jnp.dot(x_ref[...].astype(jnp.bfloat16),
                      s_ref[pl.ds(k * BK, BK), :],
                      preferred_element_type=jnp.float32)

    @pl.when(k == 0)
    def _init():
        o_ref[...] = partial

    @pl.when(k > 0)
    def _accum():
        o_ref[...] += partial


@jax.jit
def kernel(x, rand_indices, rand_signs):
    idx = rand_indices.astype(jnp.int32)
    sign = rand_signs.astype(jnp.float32)
    grid = (BATCH // BM, ORIGINAL_DIM // BK)
    return pl.pallas_call(
        _sjlt_kernel,
        grid=grid,
        in_specs=[
            pl.BlockSpec((BK, C), lambda i, k: (k, 0)),
            pl.BlockSpec((BK, C), lambda i, k: (k, 0)),
            pl.BlockSpec((BM, BK), lambda i, k: (i, k)),
        ],
        out_specs=pl.BlockSpec((BM, PROJ_DIM), lambda i, k: (i, 0)),
        out_shape=jax.ShapeDtypeStruct((BATCH, PROJ_DIM), jnp.float32),
        scratch_shapes=[pltpu.VMEM((ORIGINAL_DIM, PROJ_DIM), jnp.bfloat16)],
    )(idx, sign, x)


# speedup vs baseline: 1.4935x; 1.4935x over previous
import jax
import jax.numpy as jnp
from jax.experimental import pallas as pl
from jax.experimental.pallas import tpu as pltpu

ORIGINAL_DIM = 4096
PROJ_DIM = 1024
C = 4
BATCH = 2048

BM = 512


def _sjlt_kernel(idx_ref, sign_ref, x_ref, o_ref, s_ref):
    @pl.when(pl.program_id(0) == 0)
    def _build_s():
        DB = 512
        p = jax.lax.broadcasted_iota(jnp.int32, (DB, PROJ_DIM), 1)
        for d0 in range(0, ORIGINAL_DIM, DB):
            idx = idx_ref[d0:d0 + DB, :]
            sign = sign_ref[d0:d0 + DB, :]
            acc = jnp.zeros((DB, PROJ_DIM), jnp.float32)
            for j in range(C):
                acc += jnp.where(idx[:, j][:, None] == p,
                                 sign[:, j][:, None], 0.0)
            s_ref[d0:d0 + DB, :] = acc.astype(jnp.bfloat16)

    o_ref[...] = jnp.dot(x_ref[...].astype(jnp.bfloat16), s_ref[...],
                         preferred_element_type=jnp.float32)


@jax.jit
def kernel(x, rand_indices, rand_signs):
    idx = rand_indices.astype(jnp.int32)
    sign = rand_signs.astype(jnp.float32)
    grid = (BATCH // BM,)
    return pl.pallas_call(
        _sjlt_kernel,
        grid=grid,
        in_specs=[
            pl.BlockSpec((ORIGINAL_DIM, C), lambda i: (0, 0)),
            pl.BlockSpec((ORIGINAL_DIM, C), lambda i: (0, 0)),
            pl.BlockSpec((BM, ORIGINAL_DIM), lambda i: (i, 0)),
        ],
        out_specs=pl.BlockSpec((BM, PROJ_DIM), lambda i: (i, 0)),
        out_shape=jax.ShapeDtypeStruct((BATCH, PROJ_DIM), jnp.float32),
        scratch_shapes=[pltpu.VMEM((ORIGINAL_DIM, PROJ_DIM), jnp.bfloat16)],
    )(idx, sign, x)
